# baseline (speedup 1.0000x reference)
import numpy as np
import jax
import jax.numpy as jnp
from jax.experimental import pallas as pl
from jax.experimental.pallas import tpu as pltpu

P = 1_000_000
G = 125
BP = P // G
RQ = BP * 2 // 128

_TINY = np.float32(np.finfo(np.float32).tiny)
_LO = np.float32(np.nextafter(np.float32(-1.0), np.float32(0.0)))
_SQRT2 = np.float32(np.sqrt(2.0))


def _rotl(x, d):
    return (x << jnp.uint32(d)) | (x >> jnp.uint32(32 - d))


def _threefry2x32(k1, k2, x0, x1):
    ks2 = k1 ^ k2 ^ jnp.uint32(0x1BD11BDA)

    def rnds(x0, x1, rots):
        for r in rots:
            x0 = x0 + x1
            x1 = _rotl(x1, r)
            x1 = x1 ^ x0
        return x0, x1

    x0 = x0 + k1
    x1 = x1 + k2
    x0, x1 = rnds(x0, x1, (13, 15, 26, 6))
    x0 = x0 + k2
    x1 = x1 + ks2 + jnp.uint32(1)
    x0, x1 = rnds(x0, x1, (17, 29, 16, 24))
    x0 = x0 + ks2
    x1 = x1 + k1 + jnp.uint32(2)
    x0, x1 = rnds(x0, x1, (13, 15, 26, 6))
    x0 = x0 + k1
    x1 = x1 + k2 + jnp.uint32(3)
    x0, x1 = rnds(x0, x1, (17, 29, 16, 24))
    x0 = x0 + k2
    x1 = x1 + ks2 + jnp.uint32(4)
    x0, x1 = rnds(x0, x1, (13, 15, 26, 6))
    x0 = x0 + ks2
    x1 = x1 + k1 + jnp.uint32(5)
    return x0, x1


def _bits(k1, k2, idx):
    o0, o1 = _threefry2x32(k1, k2, jnp.zeros_like(idx), idx)
    return o0 ^ o1


def _unit(bits):
    fb = (bits >> jnp.uint32(9)) | jnp.uint32(0x3F800000)
    return jax.lax.bitcast_convert_type(fb, jnp.float32) - jnp.float32(1.0)


def _erf_inv(x):
    w = -jnp.log1p(-x * x)
    w1 = w - jnp.float32(2.5)
    p1 = jnp.full_like(x, np.float32(2.81022636e-08))
    for c in (3.43273939e-07, -3.5233877e-06, -4.39150654e-06, 0.00021858087,
              -0.00125372503, -0.00417768164, 0.246640727, 1.50140941):
        p1 = jnp.float32(c) + p1 * w1
    w2 = jnp.sqrt(jnp.maximum(w, jnp.float32(1e-30))) - jnp.float32(3.0)
    p2 = jnp.full_like(x, np.float32(-0.000200214257))
    for c in (0.000100950558, 0.00134934322, -0.00367342844, 0.00573950773,
              -0.0076224613, 0.00943887047, 1.00167406, 2.83297682):
        p2 = jnp.float32(c) + p2 * w2
    return jnp.where(w < jnp.float32(5.0), p1, p2) * x


def _bf(x):
    return x.astype(jnp.bfloat16).astype(jnp.float32)


def _mlp_kernel(d_ref, w1_ref, b1_ref, w2_ref, b2_ref, o_ref):
    eye4 = (jax.lax.broadcasted_iota(jnp.int32, (4, 4), 0)
            == jax.lax.broadcasted_iota(jnp.int32, (4, 4), 1)).astype(jnp.float32)
    p1 = _bf(w1_ref[...]) * _bf(d_ref[...])
    h = (p1[:, 0:1] + p1[:, 1:2]) + b1_ref[...]
    h = h / (jnp.float32(1.0) + jnp.abs(h))
    h_t = jnp.sum(h * eye4, axis=0, keepdims=True)
    p2 = _bf(w2_ref[...]) * _bf(h_t)
    lg = (((p2[:, 0:1] + p2[:, 1:2]) + p2[:, 2:3]) + p2[:, 3:4]) + b2_ref[...]
    m = jnp.max(lg, axis=0, keepdims=True)
    s = lg - m
    lse = jnp.log(jnp.sum(jnp.exp(s), axis=0, keepdims=True))
    o_ref[...] = s - lse


def _main_kernel(logp_ref, keys_ref, pos_ref, locs_ref, covs_ref,
                 np_ref, z_ref):
    i = pl.program_id(0)
    qbase = jnp.uint32(BP * 2) * i.astype(jnp.uint32)
    row = jax.lax.broadcasted_iota(jnp.uint32, (RQ, 128), 0)
    lane = jax.lax.broadcasted_iota(jnp.uint32, (RQ, 128), 1)
    q = qbase + row * jnp.uint32(128) + lane
    even = (lane & jnp.uint32(1)) == jnp.uint32(0)

    kz1 = keys_ref[0, 0]
    kz2 = keys_ref[0, 1]
    ke1 = keys_ref[1, 0]
    ke2 = keys_ref[1, 1]

    two_q = q * jnp.uint32(2)
    u_a = jnp.maximum(_unit(_bits(kz1, kz2, two_q)), _TINY)
    u_b = jnp.maximum(_unit(_bits(kz1, kz2, two_q + jnp.uint32(1))), _TINY)
    g_a = -jnp.log(-jnp.log(u_a))
    g_b = -jnp.log(-jnp.log(u_b))

    lp0 = logp_ref[0, 0]
    lp1 = logp_ref[1, 0]
    lp2 = logp_ref[2, 0]
    lp3 = logp_ref[3, 0]
    s_a = jnp.where(even, lp0, lp2) + g_a
    s_b = jnp.where(even, lp1, lp3) + g_b
    m = jnp.maximum(s_a, s_b)
    jloc = (s_b > s_a).astype(jnp.int32)
    m_other = jnp.where(even, jnp.roll(m, -1, axis=1), jnp.roll(m, 1, axis=1))
    j_other = jnp.where(even, jnp.roll(jloc, -1, axis=1),
                        jnp.roll(jloc, 1, axis=1))
    m_e = jnp.where(even, m, m_other)
    m_o = jnp.where(even, m_other, m)
    a_e = jnp.where(even, jloc, j_other)
    a_o = jnp.where(even, j_other, jloc)
    z = jnp.where(m_o > m_e, 2 + a_o, a_e)

    ue = jnp.maximum(_unit(_bits(ke1, ke2, q)) * jnp.float32(2.0) + _LO, _LO)
    e = _SQRT2 * _erf_inv(ue)
    e_left = jnp.roll(e, 1, axis=1)

    xl = locs_ref[0].reshape(RQ, 4, 32, 4)
    u0 = xl[:, :, :, 0].reshape(RQ, 128)
    u1 = xl[:, :, :, 1].reshape(RQ, 128)
    u2 = xl[:, :, :, 2].reshape(RQ, 128)
    u3 = xl[:, :, :, 3].reshape(RQ, 128)
    c0 = jnp.where(even, u0, jnp.roll(u1, 1, axis=1))
    c1 = jnp.where(even, u2, jnp.roll(u3, 1, axis=1))
    c2 = jnp.where(even, jnp.roll(u0, -1, axis=1), u1)
    c3 = jnp.where(even, jnp.roll(u2, -1, axis=1), u3)
    loc = jnp.where(z == 0, c0,
                    jnp.where(z == 1, c1, jnp.where(z == 2, c2, c3)))

    xc = covs_ref[0].reshape(RQ, 8, 16, 8)
    w0 = xc[:, :, :, 0].reshape(RQ, 128)
    w2 = xc[:, :, :, 2].reshape(RQ, 128)
    w3 = xc[:, :, :, 3].reshape(RQ, 128)
    w4 = xc[:, :, :, 4].reshape(RQ, 128)
    w6 = xc[:, :, :, 6].reshape(RQ, 128)
    w7 = xc[:, :, :, 7].reshape(RQ, 128)
    d0 = jnp.where(even, w0, jnp.roll(w3, 1, axis=1))
    d1 = jnp.where(even, w4, jnp.roll(w7, 1, axis=1))
    d2 = jnp.where(even, jnp.roll(w0, -1, axis=1), w3)
    d3 = jnp.where(even, jnp.roll(w4, -1, axis=1), w7)
    diag = jnp.where(z == 0, d0,
                     jnp.where(z == 1, d1, jnp.where(z == 2, d2, d3)))
    o0 = jnp.roll(w2, 1, axis=1)
    o1 = jnp.roll(w6, 1, axis=1)
    offd = jnp.where(z == 0, o0, jnp.where(z == 1, o1,
                     jnp.where(z == 2, w2, w6)))

    vel = loc + jnp.exp(diag) * e + jnp.where(even, jnp.float32(0.0),
                                              offd * e_left)
    np_ref[0] = pos_ref[0] + vel

    rsel = jax.lax.broadcasted_iota(jnp.int32, (128, 64), 0)
    csel = jax.lax.broadcasted_iota(jnp.int32, (128, 64), 1)
    half = jnp.where(rsel // 2 == csel, jnp.float32(0.5), jnp.float32(0.0))
    zf = jax.lax.dot_general(z.astype(jnp.float32), half,
                             (((1,), (0,)), ((), ())),
                             preferred_element_type=jnp.float32)
    z_ref[0] = zf.astype(jnp.int32)


def kernel(position, transition, dir_locs, dir_covs, displacement,
           W1, b1, W2, b2, t):
    t1 = jnp.asarray(t + 1, jnp.uint32).reshape(1)
    zero = jnp.zeros((1,), jnp.uint32)
    kz1, kz2 = _threefry2x32(jnp.uint32(0), jnp.uint32(42), zero, t1)
    ke1, ke2 = _threefry2x32(jnp.uint32(0), jnp.uint32(7), zero, t1)
    keys = jnp.concatenate([kz1, kz2, ke1, ke2]).reshape(2, 2)

    logp = pl.pallas_call(
        _mlp_kernel,
        out_shape=jax.ShapeDtypeStruct((4, 1), jnp.float32),
    )(displacement.reshape(1, 2), W1, b1.reshape(4, 1), W2, b2.reshape(4, 1))

    pos_v = position.reshape(G, RQ, 128)
    locs_v = dir_locs.reshape(G, RQ * 4, 128)
    covs_v = dir_covs.reshape(G, RQ * 8, 128)

    np_v, z_v = pl.pallas_call(
        _main_kernel,
        grid=(G,),
        in_specs=[
            pl.BlockSpec(memory_space=pltpu.SMEM),
            pl.BlockSpec(memory_space=pltpu.SMEM),
            pl.BlockSpec((1, RQ, 128), lambda i: (i, 0, 0)),
            pl.BlockSpec((1, RQ * 4, 128), lambda i: (i, 0, 0)),
            pl.BlockSpec((1, RQ * 8, 128), lambda i: (i, 0, 0)),
        ],
        out_specs=[
            pl.BlockSpec((1, RQ, 128), lambda i: (i, 0, 0)),
            pl.BlockSpec((1, RQ, 64), lambda i: (i, 0, 0)),
        ],
        out_shape=[
            jax.ShapeDtypeStruct((G, RQ, 128), jnp.float32),
            jax.ShapeDtypeStruct((G, RQ, 64), jnp.int32),
        ],
    )(logp, keys, pos_v, locs_v, covs_v)

    new_position = np_v.reshape(P, 2)
    z = z_v.reshape(P)
    return (new_position, z, transition, dir_locs, dir_covs)
